# baseline (speedup 1.0000x reference)
import functools

import jax
import jax.numpy as jnp
from jax import lax
from jax.experimental import pallas as pl
from jax.experimental.pallas import tpu as pltpu
from jax.experimental.pallas import tpu_sc as plsc

_CHUNK = 128


def _build_sc_kernel(B, Dx, Dy, NC, NS):
    NW = NC * NS
    b_per_w = B // NW
    n_chunks = b_per_w // _CHUNK
    mesh = plsc.VectorSubcoreMesh(core_axis_name="c", subcore_axis_name="s")

    @functools.partial(
        pl.kernel,
        mesh=mesh,
        compiler_params=pltpu.CompilerParams(use_tc_tiling_on_sc=False),
        out_type=jax.ShapeDtypeStruct((B, Dx + Dy), jnp.float32),
        scratch_types=[
            pltpu.VMEM((n_chunks, _CHUNK), jnp.int32),
            pltpu.VMEM((n_chunks, _CHUNK), jnp.int32),
            pltpu.VMEM((b_per_w, Dx), jnp.float32),
            pltpu.VMEM((b_per_w, Dy), jnp.float32),
            pltpu.SemaphoreType.DMA,
        ],
    )
    def k(x0_hbm, x1_hbm, wx_hbm, wy_hbm, out_hbm, idx0_v, idx1_v, rx_v, ry_v, sem):
        wid = lax.axis_index("s") * NC + lax.axis_index("c")
        base = wid * b_per_w
        for j in range(n_chunks):
            pltpu.sync_copy(x0_hbm.at[pl.ds(base + j * _CHUNK, _CHUNK)], idx0_v.at[j])
            pltpu.sync_copy(x1_hbm.at[pl.ds(base + j * _CHUNK, _CHUNK)], idx1_v.at[j])
        copies = []
        for j in range(n_chunks):
            copies.append(pltpu.async_copy(
                wx_hbm.at[idx0_v.at[j]], rx_v.at[pl.ds(j * _CHUNK, _CHUNK)], sem))
            copies.append(pltpu.async_copy(
                wy_hbm.at[idx1_v.at[j]], ry_v.at[pl.ds(j * _CHUNK, _CHUNK)], sem))
        for c in copies:
            c.wait()
        pltpu.sync_copy(rx_v, out_hbm.at[pl.ds(base, b_per_w), pl.ds(0, Dx)])
        pltpu.sync_copy(ry_v, out_hbm.at[pl.ds(base, b_per_w), pl.ds(Dx, Dy)])

    return k


def kernel(x, Wx, Wy):
    B = x.shape[0]
    Dx = Wx.shape[1]
    Dy = Wy.shape[1]
    info = plsc.get_sparse_core_info()
    k = _build_sc_kernel(B, Dx, Dy, info.num_cores, info.num_subcores)
    x0 = x[:, 0].astype(jnp.int32)
    x1 = x[:, 1].astype(jnp.int32)
    return k(x0, x1, Wx, Wy)
